# Initial kernel scaffold; baseline (speedup 1.0000x reference)
import jax
import jax.numpy as jnp
from jax.experimental import pallas as pl


def kernel(hidden_states, Wr0, W10, W20, Wr1, W11, W21):
    raise NotImplementedError("write your pallas kernel here")


# speedup vs baseline: 2.2036x; 2.2036x over previous
import functools

import jax
import jax.numpy as jnp
from jax import lax
from jax.experimental import pallas as pl
from jax.experimental.pallas import tpu as pltpu
from jax.experimental.pallas import tpu_sc as plsc

E = 16
K = 2
D = 768
F = 3072
T = 2048
C = 256
A = T * K
TPAD = T + 8
TRASH = A
SPAD = 4352


def _router_body(probs_ref, idx4_ref, gate2_ref):
    probs = probs_ref[...]

    lane = lax.broadcasted_iota(jnp.int32, (T, E), 1)
    m1 = jnp.max(probs, axis=1, keepdims=True)
    i1 = jnp.min(jnp.where(probs == m1, lane, E), axis=1, keepdims=True)
    oha = lane == i1
    neg = jnp.where(oha, -1.0, probs)
    m2 = jnp.max(neg, axis=1, keepdims=True)
    i2 = jnp.min(jnp.where(neg == m2, lane, E), axis=1, keepdims=True)
    ohb = lane == i2

    den = m1 + m2
    w1 = m1 / den
    w2 = m2 / den

    ohs = oha.astype(jnp.float32) + ohb.astype(jnp.float32)
    rr = lax.broadcasted_iota(jnp.int32, (512, 512), 0)
    cc = lax.broadcasted_iota(jnp.int32, (512, 512), 1)
    tri = (rr >= cc).astype(jnp.float32)
    chunks = []
    carry = jnp.zeros((1, E), jnp.float32)
    for c in range(T // 512):
        blk = ohs[c * 512:(c + 1) * 512, :]
        sc = jnp.dot(tri, blk, preferred_element_type=jnp.float32) + carry
        carry = sc[511:512, :]
        chunks.append(sc)
    s_cum = jnp.concatenate(chunks, axis=0)

    pos_a = jnp.sum(jnp.where(oha, s_cum, 0.0), axis=1,
                    keepdims=True).astype(jnp.int32) - 1
    pos_b = jnp.sum(jnp.where(ohb, s_cum, 0.0), axis=1,
                    keepdims=True).astype(jnp.int32) - 1
    keep_a = pos_a < C
    keep_b = pos_b < C
    slot_a = i1 * C + pos_a
    slot_b = i2 * C + pos_b

    total = s_cum[T - 1:T, :]
    cnt = jnp.minimum(total, float(C))
    mn = jnp.min(cnt, axis=1, keepdims=True)
    lane1 = lax.broadcasted_iota(jnp.int32, (1, E), 1)
    estar = jnp.min(jnp.where(cnt == mn, lane1, E), axis=1, keepdims=True)
    cstar = jnp.sum(jnp.where(lane1 == estar, cnt, 0.0), axis=1,
                    keepdims=True).astype(jnp.int32)
    empty = estar * C + jnp.minimum(cstar, C - 1)

    scat_a = jnp.where(keep_a, slot_a, TRASH)
    scat_b = jnp.where(keep_b, slot_b, TRASH)
    comb_a = jnp.where(keep_a, slot_a, empty)
    comb_b = jnp.where(keep_b, slot_b, empty)
    g_a = jnp.where(keep_a, w1, 0.0)
    g_b = jnp.where(keep_b, w2, 0.0)
    idx4_ref[...] = jnp.concatenate([scat_a, scat_b, comb_a, comb_b], axis=1)
    gate2_ref[...] = jnp.concatenate([g_a, g_b], axis=1)


def _router(probs):
    return pl.pallas_call(
        _router_body,
        out_shape=(
            jax.ShapeDtypeStruct((T, 4), jnp.int32),
            jax.ShapeDtypeStruct((T, 2), jnp.float32),
        ),
    )(probs)


def _mlp_body(buf_ref, w1_ref, w2_ref, gw_ref, y_ref, hs_ref):
    h = jnp.dot(buf_ref[...], w1_ref[0], preferred_element_type=jnp.float32)
    h = jax.nn.gelu(h)
    hs_ref[...] = h
    y_ref[...] = jnp.dot(h[:, 0:256], w2_ref[0, 0:256, :],
                         preferred_element_type=jnp.float32)

    def step(i, carry):
        k0 = i * 256
        y_ref[...] = y_ref[...] + jnp.dot(
            hs_ref[:, pl.ds(k0, 256)], w2_ref[0, pl.ds(k0, 256), :],
            preferred_element_type=jnp.float32)
        return carry

    lax.fori_loop(1, F // 256, step, 0)
    y_ref[...] = y_ref[...] * gw_ref[...]


def _mlp(buf, w1, w2, gw):
    return pl.pallas_call(
        _mlp_body,
        grid=(E,),
        out_shape=jax.ShapeDtypeStruct((A, D), jnp.float32),
        in_specs=[
            pl.BlockSpec((C, D), lambda e: (e, 0)),
            pl.BlockSpec((1, D, F), lambda e: (e, 0, 0)),
            pl.BlockSpec((1, F, D), lambda e: (e, 0, 0)),
            pl.BlockSpec((C, 1), lambda e: (e, 0)),
        ],
        out_specs=pl.BlockSpec((C, D), lambda e: (e, 0)),
        scratch_shapes=[pltpu.VMEM((C, F), jnp.float32)],
        compiler_params=pltpu.CompilerParams(
            dimension_semantics=("arbitrary",),
            vmem_limit_bytes=112 * 1024 * 1024,
        ),
    )(buf, w1, w2, gw)


@functools.cache
def _sc_mesh():
    return plsc.VectorSubcoreMesh(core_axis_name="c", subcore_axis_name="s")


def _dispatch_sc(xtz, scat, gates):
    @functools.partial(
        pl.kernel,
        out_type=(
            jax.ShapeDtypeStruct((A, D), jnp.float32),
            jax.ShapeDtypeStruct((A,), jnp.float32),
        ),
        mesh=_sc_mesh(),
        scratch_types=[
            pltpu.VMEM((2, 128), jnp.int32),
            pltpu.VMEM((2, 128), jnp.int32),
            pltpu.VMEM((2, 128), jnp.float32),
            pltpu.VMEM((128,), jnp.int32),
            pltpu.VMEM((128,), jnp.float32),
            pltpu.VMEM((128, D), jnp.float32),
            pltpu.VMEM((272,), jnp.int32),
            pltpu.VMEM((272,), jnp.float32),
            pltpu.VMEM_SHARED((SPAD,), jnp.int32),
            pltpu.VMEM_SHARED((SPAD,), jnp.float32),
            pltpu.SemaphoreType.DMA,
        ],
        compiler_params=pltpu.CompilerParams(needs_layout_passes=False),
    )
    def k(xtz_h, scat_h, gates_h, buf_h, gatew_h,
          sidx, svals, sgate, invv, gwv, rows, zi, zf, inv_sh, gw_sh, sem):
        cid = lax.axis_index("c")
        sid = lax.axis_index("s")
        wid = cid * 16 + sid

        zero16i = jnp.zeros((16,), jnp.int32)
        zero16f = jnp.zeros((16,), jnp.float32)
        for v in range(17):
            zi[pl.ds(v * 16, 16)] = zero16i
            zf[pl.ds(v * 16, 16)] = zero16f
        pltpu.sync_copy(zi, inv_sh.at[pl.ds(sid * 272, 272)])
        pltpu.sync_copy(zf, gw_sh.at[pl.ds(sid * 272, 272)])
        plsc.subcore_barrier()

        abase = sid * 256
        for j in range(2):
            pltpu.sync_copy(scat_h.at[pl.ds(abase + j * 128, 128)],
                            sidx.at[j])
            pltpu.sync_copy(gates_h.at[pl.ds(abase + j * 128, 128)],
                            sgate.at[j])
            for v in range(8):
                s0 = abase + j * 128 + v * 16
                tok = (lax.iota(jnp.int32, 16) + s0) // 2 + 1
                svals[j, pl.ds(v * 16, 16)] = tok
        for j in range(2):
            pltpu.sync_copy(svals.at[j], inv_sh.at[sidx.at[j]], add=True)
            pltpu.sync_copy(sgate.at[j], gw_sh.at[sidx.at[j]], add=True)
        plsc.subcore_barrier()

        gbase = wid * 128
        pltpu.sync_copy(inv_sh.at[pl.ds(gbase, 128)], invv)
        pltpu.sync_copy(gw_sh.at[pl.ds(gbase, 128)], gwv)
        pltpu.sync_copy(gwv, gatew_h.at[pl.ds(gbase, 128)])
        for v in range(8):
            w = invv[pl.ds(v * 16, 16)]
            invv[pl.ds(v * 16, 16)] = jnp.where(w == 0, T, w - 1)
        pltpu.async_copy(xtz_h.at[invv], rows, sem).wait()
        pltpu.sync_copy(rows, buf_h.at[pl.ds(gbase, 128)])

    return k(xtz, scat, gates)


def _combine_sc(y, comb):
    @functools.partial(
        pl.kernel,
        out_type=jax.ShapeDtypeStruct((TPAD, D), jnp.float32),
        mesh=_sc_mesh(),
        scratch_types=[
            pltpu.VMEM((128,), jnp.int32),
            pltpu.VMEM((64,), jnp.int32),
            pltpu.VMEM((64,), jnp.int32),
            pltpu.VMEM((64, D), jnp.float32),
            pltpu.VMEM((64, D), jnp.float32),
            pltpu.VMEM((8, D), jnp.float32),
            pltpu.SemaphoreType.DMA,
        ],
        compiler_params=pltpu.CompilerParams(needs_layout_passes=False),
    )
    def k(y_h, comb_h, out_h, cidx, ia, ib, ra, rb, zp, sem):
        cid = lax.axis_index("c")
        sid = lax.axis_index("s")
        wid = cid * 16 + sid
        abase = wid * 128
        tbase = wid * 64

        pltpu.sync_copy(comb_h.at[pl.ds(abase, 128)], cidx)
        io2 = lax.iota(jnp.int32, 16) * 2
        for v in range(4):
            ia[pl.ds(v * 16, 16)] = plsc.load_gather(cidx, [io2 + v * 32])
            ib[pl.ds(v * 16, 16)] = plsc.load_gather(cidx, [io2 + v * 32 + 1])
        pltpu.async_copy(y_h.at[ia], ra, sem).wait()
        pltpu.async_copy(y_h.at[ib], rb, sem).wait()

        def body(i, carry):
            for c in range(D // 16):
                ra[i, pl.ds(c * 16, 16)] = (ra[i, pl.ds(c * 16, 16)]
                                            + rb[i, pl.ds(c * 16, 16)])
            return carry

        lax.fori_loop(0, 64, body, 0)
        pltpu.sync_copy(ra, out_h.at[pl.ds(tbase, 64)])

        @pl.when(wid == 31)
        def _():
            zero16 = jnp.zeros((16,), jnp.float32)
            for rrow in range(8):
                for c in range(D // 16):
                    zp[rrow, pl.ds(c * 16, 16)] = zero16
            pltpu.sync_copy(zp, out_h.at[pl.ds(T, 8)])

    return k(y, comb)


def _moe_layer(xtz, wr, w1, w2):
    logits = xtz[:T] @ wr
    probs = jax.nn.softmax(logits, axis=-1)
    idx4, gate2 = _router(probs)
    scat = idx4[:, :2].reshape(A)
    comb = idx4[:, 2:].reshape(A)
    gts = gate2.reshape(A)
    buf, gatew = _dispatch_sc(xtz, scat, gts)
    y = _mlp(buf, w1, w2, gatew.reshape(A, 1))
    xtz_next = _combine_sc(y, comb)
    return xtz_next, logits


def kernel(hidden_states, Wr0, W10, W20, Wr1, W11, W21):
    x2d = hidden_states.reshape(T, D)
    xtz = jnp.concatenate(
        [x2d, jnp.zeros((TPAD - T, D), x2d.dtype)], axis=0)
    xtz, lg0 = _moe_layer(xtz, Wr0, W10, W20)
    xtz, lg1 = _moe_layer(xtz, Wr1, W11, W21)
    out = xtz[:T].reshape(1, T, D)
    return out, jnp.concatenate([lg0, lg1], axis=0)
